# baseline (speedup 1.0000x reference)
import jax
import jax.numpy as jnp
from jax import lax
from jax.experimental import pallas as pl
from jax.experimental.pallas import tpu as pltpu
from jax.experimental.pallas import tpu_sc as plsc

VOCAB = 1000000
DIM = 64
B = 16384

NUM_CORES = 2
NUM_SUBCORES = 16
LANES = 16
NW = NUM_CORES * NUM_SUBCORES
BPW = B // NW
CHUNK = 128
NCHUNK = BPW // CHUNK
WIDE = 2 * DIM


def _sc_body(th_hbm, tp_hbm, ch_hbm, cp_hbm, table_hbm, out_hbm,
             idx_t, par_t, idx_c, par_c, u0, u1, v0, v1, out_v,
             sem0, sem1):
    wid = lax.axis_index("s") * NUM_CORES + lax.axis_index("c")
    base = wid * BPW

    pltpu.sync_copy(th_hbm.at[pl.ds(base, BPW)], idx_t)
    pltpu.sync_copy(tp_hbm.at[pl.ds(base, BPW)], par_t)
    pltpu.sync_copy(ch_hbm.at[pl.ds(base, BPW)], idx_c)
    pltpu.sync_copy(cp_hbm.at[pl.ds(base, BPW)], par_c)

    ubufs, vbufs, sems = [u0, u1], [v0, v1], [sem0, sem1]

    def fire(j):
        k = j % 2
        sl = pl.ds(j * CHUNK, CHUNK)
        cu = pltpu.async_copy(table_hbm.at[idx_t.at[sl]], ubufs[k], sems[k])
        cv = pltpu.async_copy(table_hbm.at[idx_c.at[sl]], vbufs[k], sems[k])
        return cu, cv

    inflight = fire(0)

    for j in range(NCHUNK):
        cu, cv = inflight
        if j + 1 < NCHUNK:
            nxt = fire(j + 1)
        cu.wait()
        cv.wait()
        ubuf, vbuf = ubufs[j % 2], vbufs[j % 2]

        def group(g, carry):
            rows16 = lax.iota(jnp.int32, 16) + g * LANES
            sl16 = pl.ds(j * CHUNK + g * LANES, LANES)
            pu = par_t[sl16]
            pv = par_c[sl16]
            acc = jnp.zeros((LANES,), jnp.float32)
            for d in range(DIM):
                u = plsc.load_gather(ubuf, [rows16, pu + d])
                v = plsc.load_gather(vbuf, [rows16, pv + d])
                acc = acc + u * v
            out_v[sl16] = acc
            return carry

        lax.fori_loop(0, CHUNK // LANES, group, 0)
        if j + 1 < NCHUNK:
            inflight = nxt

    pltpu.sync_copy(out_v, out_hbm.at[pl.ds(base, BPW)])


@jax.jit
def _skipgram(th, tp, ch, cp, table2):
    mesh = plsc.VectorSubcoreMesh(core_axis_name="c", subcore_axis_name="s")
    return pl.kernel(
        _sc_body,
        out_type=jax.ShapeDtypeStruct((B,), jnp.float32),
        mesh=mesh,
        scratch_types=[
            pltpu.VMEM((BPW,), jnp.int32),
            pltpu.VMEM((BPW,), jnp.int32),
            pltpu.VMEM((BPW,), jnp.int32),
            pltpu.VMEM((BPW,), jnp.int32),
            pltpu.VMEM((CHUNK, WIDE), jnp.float32),
            pltpu.VMEM((CHUNK, WIDE), jnp.float32),
            pltpu.VMEM((CHUNK, WIDE), jnp.float32),
            pltpu.VMEM((CHUNK, WIDE), jnp.float32),
            pltpu.VMEM((BPW,), jnp.float32),
            pltpu.SemaphoreType.DMA,
            pltpu.SemaphoreType.DMA,
        ],
        compiler_params=pltpu.CompilerParams(needs_layout_passes=False),
    )(th, tp, ch, cp, table2)


def kernel(target, context, embedding_weights):
    t32 = target.astype(jnp.int32)
    c32 = context.astype(jnp.int32)
    table2 = embedding_weights.reshape(VOCAB // 2, WIDE)
    return _skipgram(t32 >> 1, (t32 & 1) * DIM, c32 >> 1, (c32 & 1) * DIM,
                     table2)
